# Initial kernel scaffold; baseline (speedup 1.0000x reference)
import jax
import jax.numpy as jnp
from jax.experimental import pallas as pl


def kernel(pooled, spx):
    raise NotImplementedError("write your pallas kernel here")


# speedup vs baseline: 9.9585x; 9.9585x over previous
import functools

import jax
import jax.numpy as jnp
from jax import lax
from jax.experimental import pallas as pl
from jax.experimental.pallas import tpu as pltpu
from jax.experimental.pallas import tpu_sc as plsc

B, C, K = 4, 96, 1024
H = W = 384
HW = H * W
NC, NS, L = 2, 16, 16
CH = 2
CB = C // CH
PB = NS
PIX_PER_W = HW // PB
CHUNK = 512
NCHUNK = PIX_PER_W // CHUNK
GRP = CHUNK // L


def _unpool_sc(pooled_flat, spx_flat):
    mesh = plsc.VectorSubcoreMesh(core_axis_name="c", subcore_axis_name="s")

    @functools.partial(
        pl.kernel,
        mesh=mesh,
        out_type=jax.ShapeDtypeStruct((B, C, HW), jnp.float32),
        compiler_params=pltpu.CompilerParams(needs_layout_passes=False),
        scratch_types=[
            pltpu.VMEM((CB * K,), jnp.float32),
            pltpu.VMEM((CHUNK,), jnp.int32),
            pltpu.VMEM((CB, CHUNK), jnp.float32)
        ],
    )
    def unpool(pooled_hbm, spx_hbm, out_hbm, table_v, idx_v, out_v):
        ch = lax.axis_index("c")
        pb = lax.axis_index("s")
        c0 = ch * CB
        pbase = pb * PIX_PER_W

        for b in range(B):
            pltpu.sync_copy(pooled_hbm.at[b, pl.ds(c0 * K, CB * K)], table_v)

            def chunk_body(t, carry):
                base = pbase + t * CHUNK
                pltpu.sync_copy(spx_hbm.at[b, pl.ds(base, CHUNK)], idx_v)

                def grp_body(g, carry2):
                    pix = idx_v[pl.ds(g * L, L)]
                    for c in range(CB):
                        vals = plsc.load_gather(table_v, [pix + c * K])
                        out_v[c, pl.ds(g * L, L)] = vals
                    return carry2

                lax.fori_loop(0, GRP, grp_body, 0, unroll=False)
                pltpu.sync_copy(
                    out_v, out_hbm.at[b, pl.ds(c0, CB), pl.ds(base, CHUNK)]
                )
                return carry

            lax.fori_loop(0, NCHUNK, chunk_body, 0, unroll=False)

    return unpool(pooled_flat, spx_flat)


def kernel(pooled, spx):
    pooled_flat = pooled.reshape(B, C * K)
    spx_flat = spx.reshape(B, HW)
    out = _unpool_sc(pooled_flat, spx_flat)
    return out.reshape(B, C, H, W)


# speedup vs baseline: 11.4764x; 1.1524x over previous
import functools

import jax
import jax.numpy as jnp
from jax import lax
from jax.experimental import pallas as pl
from jax.experimental.pallas import tpu as pltpu
from jax.experimental.pallas import tpu_sc as plsc

B, C, K = 4, 96, 1024
H = W = 384
HW = H * W
NC, NS, L = 2, 16, 16
CH = 2
CB = C // CH
PB = NS
PIX_PER_W = HW // PB
CHUNK = 512
NCHUNK = PIX_PER_W // CHUNK
GRP = CHUNK // L


def _unpool_sc(pooled_flat, spx_flat):
    mesh = plsc.VectorSubcoreMesh(core_axis_name="c", subcore_axis_name="s")

    @functools.partial(
        pl.kernel,
        mesh=mesh,
        out_type=jax.ShapeDtypeStruct((B, C, HW), jnp.float32),
        compiler_params=pltpu.CompilerParams(needs_layout_passes=False),
        scratch_types=[
            pltpu.VMEM((CB * K,), jnp.float32),
            pltpu.VMEM((2, CHUNK), jnp.int32),
            pltpu.VMEM((2, CB, CHUNK), jnp.float32),
            pltpu.SemaphoreType.DMA((2,)),
            pltpu.SemaphoreType.DMA((2,)),
        ],
    )
    def unpool(pooled_hbm, spx_hbm, out_hbm, table_v, idx_v, out_v, isem, osem):
        ch = lax.axis_index("c")
        pb = lax.axis_index("s")
        c0 = ch * CB
        pbase = pb * PIX_PER_W

        def idx_cp(b, t, p):
            return pltpu.make_async_copy(
                spx_hbm.at[b, pl.ds(pbase + t * CHUNK, CHUNK)],
                idx_v.at[p],
                isem.at[p],
            )

        def out_cp(b, t, p):
            return pltpu.make_async_copy(
                out_v.at[p],
                out_hbm.at[b, pl.ds(c0, CB), pl.ds(pbase + t * CHUNK, CHUNK)],
                osem.at[p],
            )

        for b in range(B):
            pltpu.sync_copy(pooled_hbm.at[b, pl.ds(c0 * K, CB * K)], table_v)
            idx_cp(b, 0, 0).start()

            def chunk_body(t, carry):
                p = t % 2

                @pl.when(t + 1 < NCHUNK)
                def _():
                    idx_cp(b, t + 1, 1 - p).start()

                idx_cp(b, t, p).wait()

                @pl.when(t >= 2)
                def _():
                    out_cp(b, t - 2, p).wait()

                def grp_body(g, carry2):
                    pix = idx_v[p, pl.ds(g * L, L)]
                    for c in range(CB):
                        vals = plsc.load_gather(table_v, [pix + c * K])
                        out_v[p, c, pl.ds(g * L, L)] = vals
                    return carry2

                lax.fori_loop(0, GRP, grp_body, 0, unroll=False)
                out_cp(b, t, p).start()
                return carry

            lax.fori_loop(0, NCHUNK, chunk_body, 0, unroll=False)
            out_cp(b, NCHUNK - 2, NCHUNK % 2).wait()
            out_cp(b, NCHUNK - 1, (NCHUNK - 1) % 2).wait()

    return unpool(pooled_flat, spx_flat)


def kernel(pooled, spx):
    pooled_flat = pooled.reshape(B, C * K)
    spx_flat = spx.reshape(B, HW)
    out = _unpool_sc(pooled_flat, spx_flat)
    return out.reshape(B, C, H, W)


# speedup vs baseline: 19.0657x; 1.6613x over previous
import functools

import jax
import jax.numpy as jnp
from jax import lax
from jax.experimental import pallas as pl
from jax.experimental.pallas import tpu as pltpu
from jax.experimental.pallas import tpu_sc as plsc

B, C, K = 4, 96, 1024
H = W = 384
HW = H * W
NC, NS, L = 2, 16, 16
CH = 2
CB = C // CH
PB = NS
PIX_PER_W = HW // PB
CHUNK = 512
NCHUNK = PIX_PER_W // CHUNK
GRP = CHUNK // L


def _unpool_sc(pooled_flat, spx_flat):
    mesh = plsc.VectorSubcoreMesh(core_axis_name="c", subcore_axis_name="s")

    @functools.partial(
        pl.kernel,
        mesh=mesh,
        out_type=jax.ShapeDtypeStruct((B, C, HW), jnp.float32),
        compiler_params=pltpu.CompilerParams(needs_layout_passes=False),
        scratch_types=[
            pltpu.VMEM((CB * K,), jnp.float32),
            pltpu.VMEM((2, CHUNK), jnp.int32),
            pltpu.VMEM((2, CB, CHUNK), jnp.float32),
            pltpu.SemaphoreType.DMA((2,)),
            pltpu.SemaphoreType.DMA((2,)),
        ],
    )
    def unpool(pooled_hbm, spx_hbm, out_hbm, table_v, idx_v, out_v, isem, osem):
        ch = lax.axis_index("c")
        pb = lax.axis_index("s")
        c0 = ch * CB
        pbase = pb * PIX_PER_W

        def idx_cp(b, t, p):
            return pltpu.make_async_copy(
                spx_hbm.at[b, pl.ds(pbase + t * CHUNK, CHUNK)],
                idx_v.at[p],
                isem.at[p],
            )

        def out_cp(b, t, p):
            return pltpu.make_async_copy(
                out_v.at[p],
                out_hbm.at[b, pl.ds(c0, CB), pl.ds(pbase + t * CHUNK, CHUNK)],
                osem.at[p],
            )

        for b in range(B):
            pltpu.sync_copy(pooled_hbm.at[b, pl.ds(c0 * K, CB * K)], table_v)
            idx_cp(b, 0, 0).start()

            def chunk_body(t, carry):
                p = t % 2

                @pl.when(t + 1 < NCHUNK)
                def _():
                    idx_cp(b, t + 1, 1 - p).start()

                idx_cp(b, t, p).wait()

                @pl.when(t >= 2)
                def _():
                    out_cp(b, t - 2, p).wait()

                def grp_body(g, carry2):
                    pix = idx_v[p, pl.ds(g * L, L)]
                    for c in range(0, CB, 8):
                        vals = [
                            plsc.load_gather(table_v, [pix + (c + j) * K])
                            for j in range(8)
                        ]
                        for j in range(8):
                            out_v[p, c + j, pl.ds(g * L, L)] = vals[j]
                    return carry2

                lax.fori_loop(0, GRP, grp_body, 0, unroll=False)
                out_cp(b, t, p).start()
                return carry

            lax.fori_loop(0, NCHUNK, chunk_body, 0, unroll=False)
            out_cp(b, NCHUNK - 2, NCHUNK % 2).wait()
            out_cp(b, NCHUNK - 1, (NCHUNK - 1) % 2).wait()

    return unpool(pooled_flat, spx_flat)


def kernel(pooled, spx):
    pooled_flat = pooled.reshape(B, C * K)
    spx_flat = spx.reshape(B, HW)
    out = _unpool_sc(pooled_flat, spx_flat)
    return out.reshape(B, C, H, W)
